# baseline (speedup 1.0000x reference)
import functools
import math

import jax
import jax.numpy as jnp
from jax import lax
from jax.experimental import pallas as pl
from jax.experimental.pallas import tpu as pltpu
from jax.experimental.pallas import tpu_sc as plsc

_T = 2048
_H = 12
_BINS = 32
_HALF = _BINS // 2
_L = 16
_NW = 32
_ROWS_PER_W = _T // _NW
_CHUNKS = _T // _L
_PI = math.pi
_TWO_PI = 2.0 * math.pi
_SPHI = _HALF / math.pi


def _rpe_sc_call(eta, phi, table):
    mesh = plsc.VectorSubcoreMesh(core_axis_name="c", subcore_axis_name="s")

    @functools.partial(
        pl.kernel,
        out_type=jax.ShapeDtypeStruct((1, _H, _T, _T), jnp.float32),
        mesh=mesh,
        scratch_types=[
            pltpu.VMEM((_T + _L,), jnp.float32),
            pltpu.VMEM((_T + _L,), jnp.float32),
            pltpu.VMEM((_BINS * 2, _H), jnp.float32),
            pltpu.VMEM((_H * 1024,), jnp.float32),
            pltpu.VMEM((2, _H, 2, _T), jnp.float32),
            pltpu.SemaphoreType.DMA,
            pltpu.SemaphoreType.DMA,
        ],
        compiler_params=pltpu.CompilerParams(needs_layout_passes=False),
    )
    def k(eta_hbm, phi_hbm, tab_hbm, out_hbm,
          eta_v, phi_v, tab_v, tf_v, buf_v, sem0, sem1):
        wid = lax.axis_index("s") * 2 + lax.axis_index("c")
        base = wid * _ROWS_PER_W

        pltpu.sync_copy(eta_hbm, eta_v.at[pl.ds(0, _T)])
        pltpu.sync_copy(phi_hbm, phi_v.at[pl.ds(0, _T)])
        pltpu.sync_copy(tab_hbm, tab_v)

        def red_body(c, carry):
            mx, mn = carry
            v = eta_v[pl.ds(c * _L, _L)]
            return jnp.maximum(mx, v), jnp.minimum(mn, v)

        v0 = eta_v[pl.ds(0, _L)]
        mx, mn = lax.fori_loop(1, _CHUNKS, red_body, (v0, v0))
        rng = jnp.maximum(
            jnp.broadcast_to(jnp.max(mx) - jnp.min(mn), (_L,)),
            jnp.float32(1e-6))
        se = jnp.float32(_HALF) / rng

        def scale_body(c, _):
            eta_v[pl.ds(c * _L, _L)] = eta_v[pl.ds(c * _L, _L)] * se
            return 0

        lax.fori_loop(0, _CHUNKS, scale_body, 0)

        def tf_body(n, _):
            v = n * _L + lax.iota(jnp.int32, _L)
            h = v >> 10
            e = (v & 1023) >> 5
            p = v & 31
            a = plsc.load_gather(tab_v, [e, h])
            b = plsc.load_gather(tab_v, [p + _BINS, h])
            tf_v[pl.ds(n * _L, _L)] = a + b
            return 0

        lax.fori_loop(0, (_H * 1024) // _L, tf_body, 0)

        def compute_pair(i0, slot):
            a_e0 = eta_v[pl.ds(i0, _L)][0]
            p_i0 = phi_v[pl.ds(i0, _L)][0]
            a_e1 = eta_v[pl.ds(i0 + 1, _L)][0]
            p_i1 = phi_v[pl.ds(i0 + 1, _L)][0]

            @plsc.parallel_loop(0, _T, step=_L, unroll=2)
            def _(off):
                ej = eta_v[pl.ds(off, _L)]
                pj = phi_v[pl.ds(off, _L)]
                for s, a_e, p_i in ((0, a_e0, p_i0), (1, a_e1, p_i1)):
                    eb = (a_e - ej).astype(jnp.int32)
                    eb = jnp.clip(eb, -_HALF, _HALF - 1)
                    d = jnp.mod(p_i - pj + _PI, _TWO_PI) - _PI
                    pb = (d * _SPHI).astype(jnp.int32)
                    pb = jnp.clip(pb, -_HALF, _HALF - 1)
                    ci = ((eb + _HALF) << 5) + (pb + _HALF)
                    for h in range(_H):
                        buf_v[slot, h, s, pl.ds(off, _L)] = plsc.load_gather(
                            tf_v, [ci + h * 1024])

        def emit_pair(i0, slot, sem):
            for h in range(_H):
                pltpu.async_copy(
                    buf_v.at[slot, h], out_hbm.at[0, h, pl.ds(i0, 2)], sem)

        def drain_pair(i0, slot, sem):
            for h in range(_H):
                pltpu.make_async_copy(
                    buf_v.at[slot, h], out_hbm.at[0, h, pl.ds(i0, 2)],
                    sem).wait()

        def quad_body(t, _):
            i0 = base + 4 * t

            @pl.when(t > 0)
            def _():
                drain_pair(i0 - 4, 0, sem0)

            compute_pair(i0, 0)
            emit_pair(i0, 0, sem0)

            @pl.when(t > 0)
            def _():
                drain_pair(i0 - 2, 1, sem1)

            compute_pair(i0 + 2, 1)
            emit_pair(i0 + 2, 1, sem1)
            return 0

        lax.fori_loop(0, _ROWS_PER_W // 4, quad_body, 0)
        drain_pair(base + _ROWS_PER_W - 4, 0, sem0)
        drain_pair(base + _ROWS_PER_W - 2, 1, sem1)

    return k(eta, phi, table)


def kernel(coords, rpe_table):
    eta = coords[0, :, 0]
    phi = coords[0, :, 1]
    return _rpe_sc_call(eta, phi, rpe_table)


# speedup vs baseline: 1.3895x; 1.3895x over previous
import functools
import math

import jax
import jax.numpy as jnp
from jax import lax
from jax.experimental import pallas as pl
from jax.experimental.pallas import tpu as pltpu
from jax.experimental.pallas import tpu_sc as plsc

_T = 2048
_H = 12
_BINS = 32
_HALF = _BINS // 2
_L = 16
_NW = 32
_ROWS_PER_W = _T // _NW
_CHUNKS = _T // _L
_PI = math.pi
_TWO_PI = 2.0 * math.pi
_SPHI = _HALF / math.pi


def _rpe_sc_call(eta, phi, table):
    mesh = plsc.VectorSubcoreMesh(core_axis_name="c", subcore_axis_name="s")

    @functools.partial(
        pl.kernel,
        out_type=jax.ShapeDtypeStruct((1, _H, _T, _T), jnp.float32),
        mesh=mesh,
        scratch_types=[
            pltpu.VMEM((_T + _L,), jnp.float32),
            pltpu.VMEM((_T + _L,), jnp.float32),
            pltpu.VMEM((_T + _L,), jnp.float32),
            pltpu.VMEM((_BINS * 2, _H), jnp.float32),
            pltpu.VMEM(((_H // 2) * 1024,), jnp.int32),
            pltpu.VMEM((2, _H, 2, _T), jnp.float32),
            pltpu.SemaphoreType.DMA,
            pltpu.SemaphoreType.DMA,
        ],
        compiler_params=pltpu.CompilerParams(needs_layout_passes=False),
    )
    def k(eta_hbm, phi_hbm, tab_hbm, out_hbm,
          eta_v, phi_v, pa_v, tab_v, tfp_v, buf_v, sem0, sem1):
        wid = lax.axis_index("s") * 2 + lax.axis_index("c")
        base = wid * _ROWS_PER_W

        pltpu.sync_copy(eta_hbm, eta_v.at[pl.ds(0, _T)])
        pltpu.sync_copy(phi_hbm, phi_v.at[pl.ds(0, _T)])
        pltpu.sync_copy(tab_hbm, tab_v)

        def red_body(c, carry):
            mx, mn = carry
            v = eta_v[pl.ds(c * _L, _L)]
            return jnp.maximum(mx, v), jnp.minimum(mn, v)

        v0 = eta_v[pl.ds(0, _L)]
        mx, mn = lax.fori_loop(1, _CHUNKS, red_body, (v0, v0))
        rng = jnp.maximum(
            jnp.broadcast_to(jnp.max(mx) - jnp.min(mn), (_L,)),
            jnp.float32(1e-6))
        se = jnp.float32(_HALF) / rng

        def scale_body(c, _):
            sl = pl.ds(c * _L, _L)
            eta_v[sl] = eta_v[sl] * se
            ph = phi_v[sl]
            pa_v[sl] = jnp.mod(ph + _PI, _TWO_PI) * _SPHI
            phi_v[sl] = jnp.mod(ph, _TWO_PI) * _SPHI
            return 0

        lax.fori_loop(0, _CHUNKS, scale_body, 0)

        def tf_body(n, _):
            v = n * _L + lax.iota(jnp.int32, _L)
            hp = v >> 10
            e = (v & 1023) >> 5
            p = (v & 31) + _BINS
            h0 = hp << 1
            h1 = h0 + 1
            s0 = plsc.load_gather(tab_v, [e, h0]) + plsc.load_gather(
                tab_v, [p, h0])
            s1 = plsc.load_gather(tab_v, [e, h1]) + plsc.load_gather(
                tab_v, [p, h1])
            b0 = plsc.bitcast(s0, jnp.int32)
            b1 = plsc.bitcast(s1, jnp.int32)
            tfp_v[pl.ds(n * _L, _L)] = ((b0 >> 16) & 0xFFFF) | (
                b1 & jnp.int32(-65536))
            return 0

        lax.fori_loop(0, ((_H // 2) * 1024) // _L, tf_body, 0)

        def compute_pair(i0, slot):
            a_e0 = eta_v[pl.ds(i0, _L)][0]
            p_i0 = pa_v[pl.ds(i0, _L)][0]
            a_e1 = eta_v[pl.ds(i0 + 1, _L)][0]
            p_i1 = pa_v[pl.ds(i0 + 1, _L)][0]

            @plsc.parallel_loop(0, _T, step=_L, unroll=1)
            def _(off):
                ej = eta_v[pl.ds(off, _L)]
                pj = phi_v[pl.ds(off, _L)]
                for s, a_e, p_i in ((0, a_e0, p_i0), (1, a_e1, p_i1)):
                    eb = (a_e - ej).astype(jnp.int32)
                    eb = jnp.minimum(eb, _HALF - 1)
                    x = p_i - pj
                    x = jnp.where(x < 0.0, x + jnp.float32(_BINS), x)
                    pb = (x - jnp.float32(_HALF)).astype(jnp.int32)
                    pb = jnp.minimum(pb, _HALF - 1)
                    ci = (eb << 5) + pb
                    for hp in range(_H // 2):
                        g = plsc.load_gather(tfp_v, [ci + (hp * 1024 + 528)])
                        buf_v[slot, 2 * hp, s, pl.ds(off, _L)] = plsc.bitcast(
                            g << 16, jnp.float32)
                        buf_v[slot, 2 * hp + 1, s, pl.ds(off, _L)] = (
                            plsc.bitcast(g & jnp.int32(-65536), jnp.float32))

        def emit_pair(i0, slot, sem):
            for h in range(_H):
                pltpu.async_copy(
                    buf_v.at[slot, h], out_hbm.at[0, h, pl.ds(i0, 2)], sem)

        def drain_pair(i0, slot, sem):
            for h in range(_H):
                pltpu.make_async_copy(
                    buf_v.at[slot, h], out_hbm.at[0, h, pl.ds(i0, 2)],
                    sem).wait()

        def quad_body(t, _):
            i0 = base + 4 * t

            @pl.when(t > 0)
            def _():
                drain_pair(i0 - 4, 0, sem0)

            compute_pair(i0, 0)
            emit_pair(i0, 0, sem0)

            @pl.when(t > 0)
            def _():
                drain_pair(i0 - 2, 1, sem1)

            compute_pair(i0 + 2, 1)
            emit_pair(i0 + 2, 1, sem1)
            return 0

        lax.fori_loop(0, _ROWS_PER_W // 4, quad_body, 0)
        drain_pair(base + _ROWS_PER_W - 4, 0, sem0)
        drain_pair(base + _ROWS_PER_W - 2, 1, sem1)

    return k(eta, phi, table)


def kernel(coords, rpe_table):
    eta = coords[0, :, 0]
    phi = coords[0, :, 1]
    return _rpe_sc_call(eta, phi, rpe_table)
